# Initial kernel scaffold; baseline (speedup 1.0000x reference)
import jax
import jax.numpy as jnp
from jax.experimental import pallas as pl


def kernel(x, edge_index, W1, b1, W2, b2, Wm, bm, Wl, bl, noise):
    raise NotImplementedError("write your pallas kernel here")


# speedup vs baseline: 5.5535x; 5.5535x over previous
import jax
import jax.numpy as jnp
from jax import lax
from jax.experimental import pallas as pl
from jax.experimental.pallas import tpu as pltpu
from jax.experimental.pallas import tpu_sc as plsc

NC = 2
NS = 16
L = 16
K = 128
D = 128
HN = 5000
PADM = 63
ACCR = 5120
_TCTILE = False


def _remap(dst_v, c, chunks):
    def rm(i, carry):
        row = i // (K // L)
        col = (i % (K // L)) * L
        dv = dst_v[row, pl.ds(col, L)]
        t = dv - c * HN
        ok = jnp.logical_and(t >= 0, t < HN)
        dst_v[row, pl.ds(col, L)] = jnp.where(ok, t, HN + (dv & PADM))
        return carry

    lax.fori_loop(0, chunks * (K // L), rm, 0)


def _seg_scatter(table, src2, dst2, chunks):
    rpt = ACCR // NS

    def body(table_ref, src_ref, dst_ref, out_ref, src_v, dst_v, bufA, bufB,
             acc, gsA, gsB, ssA, ssB):
        c = lax.axis_index("c")
        s = lax.axis_index("s")
        row0 = pl.multiple_of(s * chunks, 8)
        pltpu.sync_copy(src_ref.at[pl.ds(row0, chunks)], src_v)
        pltpu.sync_copy(dst_ref.at[pl.ds(row0, chunks)], dst_v)
        _remap(dst_v, c, chunks)

        zv = jnp.zeros((L,), jnp.float32)

        def zrow(i, carry):
            for j in range(D // L):
                bufA[i, pl.ds(j * L, L)] = zv
            return carry

        lax.fori_loop(0, K, zrow, 0)
        base = s * rpt
        nfull = rpt // K
        for kb in range(nfull):
            pltpu.sync_copy(bufA, acc.at[pl.ds(base + kb * K, K)])
        tail = rpt - nfull * K
        if tail:
            pltpu.sync_copy(bufA.at[pl.ds(0, tail)],
                            acc.at[pl.ds(base + nfull * K, tail)])
        plsc.subcore_barrier()

        def gather(j, buf, sem):
            pltpu.async_copy(table_ref.at[src_v.at[j]], buf, sem)

        def gwait(j, buf, sem):
            pltpu.make_async_copy(table_ref.at[src_v.at[j]], buf, sem).wait()

        def scat(j, buf, sem):
            pltpu.async_copy(buf, acc.at[dst_v.at[j]], sem, add=True)

        def swait(j, buf, sem):
            pltpu.make_async_copy(buf, acc.at[dst_v.at[j]], sem).wait()

        gather(0, bufA, gsA)
        gather(1, bufB, gsB)

        def step(t, carry):
            jA = 2 * t
            jB = jA + 1
            gwait(jA, bufA, gsA)
            scat(jA, bufA, ssA)
            gwait(jB, bufB, gsB)
            scat(jB, bufB, ssB)
            swait(jA, bufA, ssA)
            gather(jA + 2, bufA, gsA)
            swait(jB, bufB, ssB)
            gather(jB + 2, bufB, gsB)
            return carry

        lax.fori_loop(0, chunks // 2 - 1, step, 0)
        jA = chunks - 2
        jB = chunks - 1
        gwait(jA, bufA, gsA)
        scat(jA, bufA, ssA)
        gwait(jB, bufB, gsB)
        scat(jB, bufB, ssB)
        swait(jA, bufA, ssA)
        swait(jB, bufB, ssB)
        plsc.subcore_barrier()
        pltpu.sync_copy(acc.at[pl.ds(base, rpt)],
                        out_ref.at[c, pl.ds(base, rpt)])

    return pl.kernel(
        body,
        out_type=jax.ShapeDtypeStruct((NC, ACCR, D), jnp.float32),
        compiler_params=pltpu.CompilerParams(use_tc_tiling_on_sc=_TCTILE),
        mesh=plsc.VectorSubcoreMesh(core_axis_name="c", subcore_axis_name="s"),
        scratch_types=[
            pltpu.VMEM((chunks, K), jnp.int32),
            pltpu.VMEM((chunks, K), jnp.int32),
            pltpu.VMEM((K, D), jnp.float32),
            pltpu.VMEM((K, D), jnp.float32),
            pltpu.VMEM_SHARED((ACCR, D), jnp.float32),
            pltpu.SemaphoreType.DMA,
            pltpu.SemaphoreType.DMA,
            pltpu.SemaphoreType.DMA,
            pltpu.SemaphoreType.DMA,
        ],
    )(table, src2, dst2)


def _degrees(dst2, chunks):
    rpt = ACCR // NS

    def body(dst_ref, out_ref, dst_v, ones_v, zbuf, vtmp, accd):
        c = lax.axis_index("c")
        s = lax.axis_index("s")
        row0 = pl.multiple_of(s * chunks, 8)
        pltpu.sync_copy(dst_ref.at[pl.ds(row0, chunks)], dst_v)
        _remap(dst_v, c, chunks)
        ov = jnp.ones((L,), jnp.float32)
        zv = jnp.zeros((L,), jnp.float32)

        def fill(i, carry):
            ones_v[pl.ds(i * L, L)] = ov
            zbuf[pl.ds(i * L, L)] = zv
            return carry

        lax.fori_loop(0, K // L, fill, 0)
        base = s * rpt
        nfull = rpt // K
        for kb in range(nfull):
            pltpu.sync_copy(zbuf, accd.at[pl.ds(base + kb * K, K)])
        tail = rpt - nfull * K
        if tail:
            pltpu.sync_copy(zbuf.at[pl.ds(0, tail)],
                            accd.at[pl.ds(base + nfull * K, tail)])
        plsc.subcore_barrier()

        def step(j, carry):
            pltpu.sync_copy(ones_v, accd.at[dst_v.at[j]], add=True)
            return carry

        lax.fori_loop(0, chunks, step, 0)
        plsc.subcore_barrier()
        pltpu.sync_copy(accd.at[pl.ds(base, rpt)], vtmp)
        pltpu.sync_copy(vtmp, out_ref.at[pl.ds(c * ACCR + base, rpt)])

    return pl.kernel(
        body,
        out_type=jax.ShapeDtypeStruct((NC * ACCR,), jnp.float32),
        compiler_params=pltpu.CompilerParams(use_tc_tiling_on_sc=_TCTILE),
        mesh=plsc.VectorSubcoreMesh(core_axis_name="c", subcore_axis_name="s"),
        scratch_types=[
            pltpu.VMEM((chunks, K), jnp.int32),
            pltpu.VMEM((K,), jnp.float32),
            pltpu.VMEM((K,), jnp.float32),
            pltpu.VMEM((rpt,), jnp.float32),
            pltpu.VMEM_SHARED((ACCR,), jnp.float32),
        ],
    )(dst2)


_BR = 1000


def _sspec(width):
    nb = HN // _BR
    return pl.BlockSpec((1, _BR, width), lambda i: (i // nb, i % nb, 0))


def _tc1(deg, x, W1):
    N, Din = x.shape
    H = W1.shape[1]

    def body(deg_ref, x_ref, w_ref, m_ref, dinv_ref):
        d = deg_ref[0] + 1.0
        dinv = lax.rsqrt(d)
        u = x_ref[...] * dinv
        m_ref[...] = jnp.dot(u, w_ref[...], preferred_element_type=jnp.float32)
        dinv_ref[...] = dinv

    return pl.pallas_call(
        body,
        grid=(N // _BR,),
        in_specs=[
            _sspec(1),
            pl.BlockSpec((_BR, Din), lambda i: (i, 0)),
            pl.BlockSpec((Din, H), lambda i: (0, 0)),
        ],
        out_specs=[
            pl.BlockSpec((_BR, H), lambda i: (i, 0)),
            pl.BlockSpec((_BR, 1), lambda i: (i, 0)),
        ],
        out_shape=[
            jax.ShapeDtypeStruct((N, H), jnp.float32),
            jax.ShapeDtypeStruct((N, 1), jnp.float32),
        ],
    )(deg, x, W1)


def _tc_mid(s, m, dinv, b, W):
    N, H = m.shape
    H2 = W.shape[1]

    def body(s_ref, m_ref, dinv_ref, b_ref, w_ref, out_ref):
        t = s_ref[0] + m_ref[...]
        h = jnp.tanh(dinv_ref[...] * t + b_ref[...])
        out_ref[...] = jnp.dot(dinv_ref[...] * h, w_ref[...],
                               preferred_element_type=jnp.float32)

    return pl.pallas_call(
        body,
        grid=(N // _BR,),
        in_specs=[
            _sspec(H),
            pl.BlockSpec((_BR, H), lambda i: (i, 0)),
            pl.BlockSpec((_BR, 1), lambda i: (i, 0)),
            pl.BlockSpec((1, H), lambda i: (0, 0)),
            pl.BlockSpec((H, H2), lambda i: (0, 0)),
        ],
        out_specs=pl.BlockSpec((_BR, H2), lambda i: (i, 0)),
        out_shape=jax.ShapeDtypeStruct((N, H2), jnp.float32),
    )(s, m, dinv, b, W)


def _tc3(s, m, dinv, b):
    N, H = m.shape

    def body(s_ref, m_ref, dinv_ref, b_ref, out_ref):
        t = s_ref[0] + m_ref[...]
        h = jnp.tanh(dinv_ref[...] * t + b_ref[...])
        out_ref[...] = dinv_ref[...] * h

    return pl.pallas_call(
        body,
        grid=(N // _BR,),
        in_specs=[
            _sspec(H),
            pl.BlockSpec((_BR, H), lambda i: (i, 0)),
            pl.BlockSpec((_BR, 1), lambda i: (i, 0)),
            pl.BlockSpec((1, H), lambda i: (0, 0)),
        ],
        out_specs=pl.BlockSpec((_BR, H), lambda i: (i, 0)),
        out_shape=jax.ShapeDtypeStruct((N, H), jnp.float32),
    )(s, m, dinv, b)


def _tc4(s, u3, dinv, Wm, bm, Wl, bl, noise):
    N, H = u3.shape
    LD = Wm.shape[1]

    def body(s_ref, u_ref, dinv_ref, wm_ref, bm_ref, wl_ref, bl_ref, n_ref,
             z_ref, mean_ref, lv_ref):
        g = dinv_ref[...] * (s_ref[0] + u_ref[...])
        mean = jnp.dot(g, wm_ref[...], preferred_element_type=jnp.float32)
        mean = mean + bm_ref[...]
        lv = jnp.dot(g, wl_ref[...], preferred_element_type=jnp.float32)
        lv = lv + bl_ref[...]
        z_ref[...] = n_ref[...] * jnp.exp(0.5 * lv) + mean
        mean_ref[...] = mean
        lv_ref[...] = lv

    return pl.pallas_call(
        body,
        grid=(N // _BR,),
        in_specs=[
            _sspec(H),
            pl.BlockSpec((_BR, H), lambda i: (i, 0)),
            pl.BlockSpec((_BR, 1), lambda i: (i, 0)),
            pl.BlockSpec((H, LD), lambda i: (0, 0)),
            pl.BlockSpec((1, LD), lambda i: (0, 0)),
            pl.BlockSpec((H, LD), lambda i: (0, 0)),
            pl.BlockSpec((1, LD), lambda i: (0, 0)),
            pl.BlockSpec((_BR, LD), lambda i: (i, 0)),
        ],
        out_specs=[
            pl.BlockSpec((_BR, LD), lambda i: (i, 0)),
            pl.BlockSpec((_BR, LD), lambda i: (i, 0)),
            pl.BlockSpec((_BR, LD), lambda i: (i, 0)),
        ],
        out_shape=[
            jax.ShapeDtypeStruct((N, LD), jnp.float32),
            jax.ShapeDtypeStruct((N, LD), jnp.float32),
            jax.ShapeDtypeStruct((N, LD), jnp.float32),
        ],
    )(s, u3, dinv, Wm, bm, Wl, bl, noise)


def kernel(x, edge_index, W1, b1, W2, b2, Wm, bm, Wl, bl, noise):
    N = x.shape[0]
    E = edge_index.shape[1]
    src = edge_index[0].astype(jnp.int32)
    dst = edge_index[1].astype(jnp.int32)

    chunks = -(-E // (NS * K))
    chunks = -(-chunks // 8) * 8
    epad = NS * chunks * K
    src2 = jnp.concatenate(
        [src, jnp.zeros((epad - E,), jnp.int32)]).reshape(NS * chunks, K)
    dst2 = jnp.concatenate(
        [dst, jnp.full((epad - E,), N, jnp.int32)]).reshape(NS * chunks, K)

    deg = _degrees(dst2, chunks).reshape(NC, ACCR, 1)
    m1, dinv = _tc1(deg, x, W1)
    s1 = _seg_scatter(m1, src2, dst2, chunks)
    m2 = _tc_mid(s1, m1, dinv, b1.reshape(1, -1), W2)
    s2 = _seg_scatter(m2, src2, dst2, chunks)
    u3 = _tc3(s2, m2, dinv, b2.reshape(1, -1))
    s3 = _seg_scatter(u3, src2, dst2, chunks)
    z, mean, logvar = _tc4(s3, u3, dinv, Wm, bm.reshape(1, -1),
                           Wl, bl.reshape(1, -1), noise)
    return (z, mean, logvar)


# speedup vs baseline: 5.8047x; 1.0452x over previous
import jax
import jax.numpy as jnp
from jax import lax
from jax.experimental import pallas as pl
from jax.experimental.pallas import tpu as pltpu
from jax.experimental.pallas import tpu_sc as plsc

NC = 2
NS = 16
L = 16
NW = NC * NS
K = 64
D = 128
NB = 4
HN = 5000
PADM = 63
ACCR = 5120
_TCTILE = False


def _remap(dst_v, c, chunks):
    def rm(i, carry):
        row = i // (K // L)
        col = (i % (K // L)) * L
        dv = dst_v[row, pl.ds(col, L)]
        t = dv - c * HN
        ok = jnp.logical_and(t >= 0, t < HN)
        dst_v[row, pl.ds(col, L)] = jnp.where(ok, t, HN + (dv & PADM))
        return carry

    lax.fori_loop(0, chunks * (K // L), rm, 0)


def _seg_scatter(table, src2, dst2, chunks):
    rpt = ACCR // NS

    def body(table_ref, src_ref, dst_ref, out_ref, src_v, dst_v,
             b0, b1, b2, b3, acc, g0, g1, g2, g3, s0, s1, s2, s3):
        bufs = (b0, b1, b2, b3)
        gs = (g0, g1, g2, g3)
        ss = (s0, s1, s2, s3)
        c = lax.axis_index("c")
        s = lax.axis_index("s")
        row0 = pl.multiple_of(s * chunks, 8)
        pltpu.sync_copy(src_ref.at[pl.ds(row0, chunks)], src_v)
        pltpu.sync_copy(dst_ref.at[pl.ds(row0, chunks)], dst_v)
        _remap(dst_v, c, chunks)

        zv = jnp.zeros((L,), jnp.float32)

        def zrow(i, carry):
            for j in range(D // L):
                b0[i, pl.ds(j * L, L)] = zv
            return carry

        lax.fori_loop(0, K, zrow, 0)
        base = s * rpt
        nfull = rpt // K
        for kb in range(nfull):
            pltpu.sync_copy(b0, acc.at[pl.ds(base + kb * K, K)])
        tail = rpt - nfull * K
        if tail:
            pltpu.sync_copy(b0.at[pl.ds(0, tail)],
                            acc.at[pl.ds(base + nfull * K, tail)])
        plsc.subcore_barrier()

        def gather(j, b):
            pltpu.async_copy(table_ref.at[src_v.at[j]], bufs[b], gs[b])

        def gwait(j, b):
            pltpu.make_async_copy(
                table_ref.at[src_v.at[j]], bufs[b], gs[b]).wait()

        def scat(j, b):
            pltpu.async_copy(bufs[b], acc.at[dst_v.at[j]], ss[b], add=True)

        def swait(j, b):
            pltpu.make_async_copy(bufs[b], acc.at[dst_v.at[j]], ss[b]).wait()

        for b in range(NB):
            gather(b, b)

        def step(t, carry):
            j0 = NB * t
            for b in range(NB):
                gwait(j0 + b, b)
                scat(j0 + b, b)
            for b in range(NB):
                swait(j0 + b, b)
                jn = j0 + NB + b
                gather(lax.select(jn < chunks, jn, 0), b)
            return carry

        lax.fori_loop(0, chunks // NB, step, 0)
        for b in range(NB):
            gwait(0, b)
        plsc.subcore_barrier()
        pltpu.sync_copy(acc.at[pl.ds(base, rpt)],
                        out_ref.at[c, pl.ds(base, rpt)])

    return pl.kernel(
        body,
        out_type=jax.ShapeDtypeStruct((NC, ACCR, D), jnp.float32),
        compiler_params=pltpu.CompilerParams(use_tc_tiling_on_sc=_TCTILE),
        mesh=plsc.VectorSubcoreMesh(core_axis_name="c", subcore_axis_name="s"),
        scratch_types=[
            pltpu.VMEM((chunks, K), jnp.int32),
            pltpu.VMEM((chunks, K), jnp.int32),
            pltpu.VMEM((K, D), jnp.float32),
            pltpu.VMEM((K, D), jnp.float32),
            pltpu.VMEM((K, D), jnp.float32),
            pltpu.VMEM((K, D), jnp.float32),
            pltpu.VMEM_SHARED((ACCR, D), jnp.float32),
            pltpu.SemaphoreType.DMA,
            pltpu.SemaphoreType.DMA,
            pltpu.SemaphoreType.DMA,
            pltpu.SemaphoreType.DMA,
            pltpu.SemaphoreType.DMA,
            pltpu.SemaphoreType.DMA,
            pltpu.SemaphoreType.DMA,
            pltpu.SemaphoreType.DMA,
        ],
    )(table, src2, dst2)


def _degrees(dst2, chunks):
    rpt = ACCR // NS

    def body(dst_ref, out_ref, dst_v, ones_v, zbuf, vtmp, accd):
        c = lax.axis_index("c")
        s = lax.axis_index("s")
        row0 = pl.multiple_of(s * chunks, 8)
        pltpu.sync_copy(dst_ref.at[pl.ds(row0, chunks)], dst_v)
        _remap(dst_v, c, chunks)
        ov = jnp.ones((L,), jnp.float32)
        zv = jnp.zeros((L,), jnp.float32)

        def fill(i, carry):
            ones_v[pl.ds(i * L, L)] = ov
            zbuf[pl.ds(i * L, L)] = zv
            return carry

        lax.fori_loop(0, K // L, fill, 0)
        base = s * rpt
        nfull = rpt // K
        for kb in range(nfull):
            pltpu.sync_copy(zbuf, accd.at[pl.ds(base + kb * K, K)])
        tail = rpt - nfull * K
        if tail:
            pltpu.sync_copy(zbuf.at[pl.ds(0, tail)],
                            accd.at[pl.ds(base + nfull * K, tail)])
        plsc.subcore_barrier()

        def step(j, carry):
            pltpu.sync_copy(ones_v, accd.at[dst_v.at[j]], add=True)
            return carry

        lax.fori_loop(0, chunks, step, 0)
        plsc.subcore_barrier()
        pltpu.sync_copy(accd.at[pl.ds(base, rpt)], vtmp)
        pltpu.sync_copy(vtmp, out_ref.at[pl.ds(c * ACCR + base, rpt)])

    return pl.kernel(
        body,
        out_type=jax.ShapeDtypeStruct((NC * ACCR,), jnp.float32),
        compiler_params=pltpu.CompilerParams(use_tc_tiling_on_sc=_TCTILE),
        mesh=plsc.VectorSubcoreMesh(core_axis_name="c", subcore_axis_name="s"),
        scratch_types=[
            pltpu.VMEM((chunks, K), jnp.int32),
            pltpu.VMEM((K,), jnp.float32),
            pltpu.VMEM((K,), jnp.float32),
            pltpu.VMEM((rpt,), jnp.float32),
            pltpu.VMEM_SHARED((ACCR,), jnp.float32),
        ],
    )(dst2)


_BR = 1000


def _sspec(width):
    nb = HN // _BR
    return pl.BlockSpec((1, _BR, width), lambda i: (i // nb, i % nb, 0))


def _tc1(deg, x, W1):
    N, Din = x.shape
    H = W1.shape[1]

    def body(deg_ref, x_ref, w_ref, m_ref, dinv_ref):
        d = deg_ref[0] + 1.0
        dinv = lax.rsqrt(d)
        u = x_ref[...] * dinv
        m_ref[...] = jnp.dot(u, w_ref[...], preferred_element_type=jnp.float32)
        dinv_ref[...] = dinv

    return pl.pallas_call(
        body,
        grid=(N // _BR,),
        in_specs=[
            _sspec(1),
            pl.BlockSpec((_BR, Din), lambda i: (i, 0)),
            pl.BlockSpec((Din, H), lambda i: (0, 0)),
        ],
        out_specs=[
            pl.BlockSpec((_BR, H), lambda i: (i, 0)),
            pl.BlockSpec((_BR, 1), lambda i: (i, 0)),
        ],
        out_shape=[
            jax.ShapeDtypeStruct((N, H), jnp.float32),
            jax.ShapeDtypeStruct((N, 1), jnp.float32),
        ],
    )(deg, x, W1)


def _tc_mid(s, m, dinv, b, W):
    N, H = m.shape
    H2 = W.shape[1]

    def body(s_ref, m_ref, dinv_ref, b_ref, w_ref, out_ref):
        t = s_ref[0] + m_ref[...]
        h = jnp.tanh(dinv_ref[...] * t + b_ref[...])
        out_ref[...] = jnp.dot(dinv_ref[...] * h, w_ref[...],
                               preferred_element_type=jnp.float32)

    return pl.pallas_call(
        body,
        grid=(N // _BR,),
        in_specs=[
            _sspec(H),
            pl.BlockSpec((_BR, H), lambda i: (i, 0)),
            pl.BlockSpec((_BR, 1), lambda i: (i, 0)),
            pl.BlockSpec((1, H), lambda i: (0, 0)),
            pl.BlockSpec((H, H2), lambda i: (0, 0)),
        ],
        out_specs=pl.BlockSpec((_BR, H2), lambda i: (i, 0)),
        out_shape=jax.ShapeDtypeStruct((N, H2), jnp.float32),
    )(s, m, dinv, b, W)


def _tc3(s, m, dinv, b):
    N, H = m.shape

    def body(s_ref, m_ref, dinv_ref, b_ref, out_ref):
        t = s_ref[0] + m_ref[...]
        h = jnp.tanh(dinv_ref[...] * t + b_ref[...])
        out_ref[...] = dinv_ref[...] * h

    return pl.pallas_call(
        body,
        grid=(N // _BR,),
        in_specs=[
            _sspec(H),
            pl.BlockSpec((_BR, H), lambda i: (i, 0)),
            pl.BlockSpec((_BR, 1), lambda i: (i, 0)),
            pl.BlockSpec((1, H), lambda i: (0, 0)),
        ],
        out_specs=pl.BlockSpec((_BR, H), lambda i: (i, 0)),
        out_shape=jax.ShapeDtypeStruct((N, H), jnp.float32),
    )(s, m, dinv, b)


def _tc4(s, u3, dinv, Wm, bm, Wl, bl, noise):
    N, H = u3.shape
    LD = Wm.shape[1]

    def body(s_ref, u_ref, dinv_ref, wm_ref, bm_ref, wl_ref, bl_ref, n_ref,
             z_ref, mean_ref, lv_ref):
        g = dinv_ref[...] * (s_ref[0] + u_ref[...])
        mean = jnp.dot(g, wm_ref[...], preferred_element_type=jnp.float32)
        mean = mean + bm_ref[...]
        lv = jnp.dot(g, wl_ref[...], preferred_element_type=jnp.float32)
        lv = lv + bl_ref[...]
        z_ref[...] = n_ref[...] * jnp.exp(0.5 * lv) + mean
        mean_ref[...] = mean
        lv_ref[...] = lv

    return pl.pallas_call(
        body,
        grid=(N // _BR,),
        in_specs=[
            _sspec(H),
            pl.BlockSpec((_BR, H), lambda i: (i, 0)),
            pl.BlockSpec((_BR, 1), lambda i: (i, 0)),
            pl.BlockSpec((H, LD), lambda i: (0, 0)),
            pl.BlockSpec((1, LD), lambda i: (0, 0)),
            pl.BlockSpec((H, LD), lambda i: (0, 0)),
            pl.BlockSpec((1, LD), lambda i: (0, 0)),
            pl.BlockSpec((_BR, LD), lambda i: (i, 0)),
        ],
        out_specs=[
            pl.BlockSpec((_BR, LD), lambda i: (i, 0)),
            pl.BlockSpec((_BR, LD), lambda i: (i, 0)),
            pl.BlockSpec((_BR, LD), lambda i: (i, 0)),
        ],
        out_shape=[
            jax.ShapeDtypeStruct((N, LD), jnp.float32),
            jax.ShapeDtypeStruct((N, LD), jnp.float32),
            jax.ShapeDtypeStruct((N, LD), jnp.float32),
        ],
    )(s, u3, dinv, Wm, bm, Wl, bl, noise)


def kernel(x, edge_index, W1, b1, W2, b2, Wm, bm, Wl, bl, noise):
    N = x.shape[0]
    E = edge_index.shape[1]
    src = edge_index[0].astype(jnp.int32)
    dst = edge_index[1].astype(jnp.int32)

    chunks = -(-E // (NS * K))
    chunks = -(-chunks // 8) * 8
    epad = NS * chunks * K
    src2 = jnp.concatenate(
        [src, jnp.zeros((epad - E,), jnp.int32)]).reshape(NS * chunks, K)
    dst2 = jnp.concatenate(
        [dst, jnp.full((epad - E,), N, jnp.int32)]).reshape(NS * chunks, K)

    deg = _degrees(dst2, chunks).reshape(NC, ACCR, 1)
    m1, dinv = _tc1(deg, x, W1)
    s1 = _seg_scatter(m1, src2, dst2, chunks)
    m2 = _tc_mid(s1, m1, dinv, b1.reshape(1, -1), W2)
    s2 = _seg_scatter(m2, src2, dst2, chunks)
    u3 = _tc3(s2, m2, dinv, b2.reshape(1, -1))
    s3 = _seg_scatter(u3, src2, dst2, chunks)
    z, mean, logvar = _tc4(s3, u3, dinv, Wm, bm.reshape(1, -1),
                           Wl, bl.reshape(1, -1), noise)
    return (z, mean, logvar)


# speedup vs baseline: 27.3718x; 4.7154x over previous
import jax
import jax.numpy as jnp
from jax import lax
from jax.experimental import pallas as pl
from jax.experimental.pallas import tpu as pltpu
from jax.experimental.pallas import tpu_sc as plsc

NC = 2
NS = 16
L = 16
NW = NC * NS
K = 64
D = 128
NB = 4
HN = 5000
PADM = 63
ACCR = 5120
_TCTILE = False


def _remap_flat(dst_v, c, n):
    def rm(i, carry):
        dv = dst_v[pl.ds(i * L, L)]
        t = dv - c * HN
        ok = jnp.logical_and(t >= 0, t < HN)
        dst_v[pl.ds(i * L, L)] = jnp.where(ok, t, HN + (dv & PADM))
        return carry

    lax.fori_loop(0, n // L, rm, 0)


def _seg_scatter(table, src2, dst2, chunks):
    rpt = ACCR // NS

    def body(table_ref, src_ref, dst_ref, out_ref, src_v, dst_v,
             b0, b1, b2, b3, acc, g0, g1, g2, g3, s0, s1, s2, s3):
        bufs = (b0, b1, b2, b3)
        gs = (g0, g1, g2, g3)
        ss = (s0, s1, s2, s3)
        c = lax.axis_index("c")
        s = lax.axis_index("s")
        ew = chunks * K
        e0 = pl.multiple_of(s * ew, 8)
        pltpu.sync_copy(src_ref.at[pl.ds(e0, ew)], src_v.at[pl.ds(0, ew)])
        pltpu.sync_copy(dst_ref.at[pl.ds(e0, ew)], dst_v.at[pl.ds(0, ew)])

        def cpt(g, off):
            sv = src_v[pl.ds(g * L, L)]
            dv = dst_v[pl.ds(g * L, L)]
            t = dv - c * HN
            ok = jnp.logical_and(t >= 0, t < HN)
            plsc.store_compressed(src_v.at[pl.ds(off, L)], sv, mask=ok)
            plsc.store_compressed(dst_v.at[pl.ds(off, L)], t, mask=ok)
            return off + plsc.all_reduce_population_count(ok)[0]

        off = lax.fori_loop(0, ew // L, cpt, jnp.int32(0))
        pv = jnp.full((L,), HN, dtype=jnp.int32)

        def pad(g, carry):
            dst_v[pl.ds(off + g * L, L)] = pv
            return carry

        lax.fori_loop(0, NB * K // L, pad, 0)
        nc = lax.max((off + K - 1) // K, 1)

        zv = jnp.zeros((L,), jnp.float32)

        def zrow(i, carry):
            for j in range(D // L):
                b0[i, pl.ds(j * L, L)] = zv
            return carry

        lax.fori_loop(0, K, zrow, 0)
        base = s * rpt
        nfull = rpt // K
        for kb in range(nfull):
            pltpu.sync_copy(b0, acc.at[pl.ds(base + kb * K, K)])
        tail = rpt - nfull * K
        if tail:
            pltpu.sync_copy(b0.at[pl.ds(0, tail)],
                            acc.at[pl.ds(base + nfull * K, tail)])
        plsc.subcore_barrier()

        def gather(j, b):
            pltpu.async_copy(
                table_ref.at[src_v.at[pl.ds(j * K, K)]], bufs[b], gs[b])

        def gwait(j, b):
            pltpu.make_async_copy(
                table_ref.at[src_v.at[pl.ds(j * K, K)]], bufs[b], gs[b]).wait()

        def scat(j, b):
            pltpu.async_copy(
                bufs[b], acc.at[dst_v.at[pl.ds(j * K, K)]], ss[b], add=True)

        def swait(j, b):
            pltpu.make_async_copy(
                bufs[b], acc.at[dst_v.at[pl.ds(j * K, K)]], ss[b]).wait()

        for b in range(NB):
            gather(b, b)

        to = (nc + NB - 1) // NB

        def step(t, carry):
            j0 = NB * t
            for b in range(NB):
                gwait(j0 + b, b)
                scat(j0 + b, b)
            for b in range(NB):
                swait(j0 + b, b)
                jn = j0 + NB + b
                gather(lax.select(jn < NB * to, jn, 0), b)
            return carry

        lax.fori_loop(0, to, step, 0)
        for b in range(NB):
            gwait(0, b)
        plsc.subcore_barrier()
        pltpu.sync_copy(acc.at[pl.ds(base, rpt)],
                        out_ref.at[c, pl.ds(base, rpt)])

    return pl.kernel(
        body,
        out_type=jax.ShapeDtypeStruct((NC, ACCR, D), jnp.float32),
        compiler_params=pltpu.CompilerParams(
            use_tc_tiling_on_sc=_TCTILE, needs_layout_passes=False),
        mesh=plsc.VectorSubcoreMesh(core_axis_name="c", subcore_axis_name="s"),
        scratch_types=[
            pltpu.VMEM(((chunks + NB) * K,), jnp.int32),
            pltpu.VMEM(((chunks + NB) * K,), jnp.int32),
            pltpu.VMEM((K, D), jnp.float32),
            pltpu.VMEM((K, D), jnp.float32),
            pltpu.VMEM((K, D), jnp.float32),
            pltpu.VMEM((K, D), jnp.float32),
            pltpu.VMEM_SHARED((ACCR, D), jnp.float32),
            pltpu.SemaphoreType.DMA,
            pltpu.SemaphoreType.DMA,
            pltpu.SemaphoreType.DMA,
            pltpu.SemaphoreType.DMA,
            pltpu.SemaphoreType.DMA,
            pltpu.SemaphoreType.DMA,
            pltpu.SemaphoreType.DMA,
            pltpu.SemaphoreType.DMA,
        ],
    )(table, src2, dst2)


def _degrees(dst2, chunks):
    rpt = ACCR // NS

    def body(dst_ref, out_ref, dst_v, ones_v, zbuf, vtmp, accd):
        c = lax.axis_index("c")
        s = lax.axis_index("s")
        ew = chunks * K
        e0 = pl.multiple_of(s * ew, 8)
        pltpu.sync_copy(dst_ref.at[pl.ds(e0, ew)], dst_v)
        _remap_flat(dst_v, c, ew)
        ov = jnp.ones((L,), jnp.float32)
        zv = jnp.zeros((L,), jnp.float32)

        def fill(i, carry):
            ones_v[pl.ds(i * L, L)] = ov
            zbuf[pl.ds(i * L, L)] = zv
            return carry

        lax.fori_loop(0, K // L, fill, 0)
        base = s * rpt
        nfull = rpt // K
        for kb in range(nfull):
            pltpu.sync_copy(zbuf, accd.at[pl.ds(base + kb * K, K)])
        tail = rpt - nfull * K
        if tail:
            pltpu.sync_copy(zbuf.at[pl.ds(0, tail)],
                            accd.at[pl.ds(base + nfull * K, tail)])
        plsc.subcore_barrier()

        def step(j, carry):
            pltpu.sync_copy(ones_v, accd.at[dst_v.at[pl.ds(j * K, K)]],
                            add=True)
            return carry

        lax.fori_loop(0, chunks, step, 0)
        plsc.subcore_barrier()
        pltpu.sync_copy(accd.at[pl.ds(base, rpt)], vtmp)
        pltpu.sync_copy(vtmp, out_ref.at[pl.ds(c * ACCR + base, rpt)])

    return pl.kernel(
        body,
        out_type=jax.ShapeDtypeStruct((NC * ACCR,), jnp.float32),
        compiler_params=pltpu.CompilerParams(use_tc_tiling_on_sc=_TCTILE),
        mesh=plsc.VectorSubcoreMesh(core_axis_name="c", subcore_axis_name="s"),
        scratch_types=[
            pltpu.VMEM((chunks * K,), jnp.int32),
            pltpu.VMEM((K,), jnp.float32),
            pltpu.VMEM((K,), jnp.float32),
            pltpu.VMEM((rpt,), jnp.float32),
            pltpu.VMEM_SHARED((ACCR,), jnp.float32),
        ],
    )(dst2)


_BR = 1000


def _sspec(width):
    nb = HN // _BR
    return pl.BlockSpec((1, _BR, width), lambda i: (i // nb, i % nb, 0))


def _tc1(deg, x, W1):
    N, Din = x.shape
    H = W1.shape[1]

    def body(deg_ref, x_ref, w_ref, m_ref, dinv_ref):
        d = deg_ref[0] + 1.0
        dinv = lax.rsqrt(d)
        u = x_ref[...] * dinv
        m_ref[...] = jnp.dot(u, w_ref[...], preferred_element_type=jnp.float32)
        dinv_ref[...] = dinv

    return pl.pallas_call(
        body,
        grid=(N // _BR,),
        in_specs=[
            _sspec(1),
            pl.BlockSpec((_BR, Din), lambda i: (i, 0)),
            pl.BlockSpec((Din, H), lambda i: (0, 0)),
        ],
        out_specs=[
            pl.BlockSpec((_BR, H), lambda i: (i, 0)),
            pl.BlockSpec((_BR, 1), lambda i: (i, 0)),
        ],
        out_shape=[
            jax.ShapeDtypeStruct((N, H), jnp.float32),
            jax.ShapeDtypeStruct((N, 1), jnp.float32),
        ],
    )(deg, x, W1)


def _tc_mid(s, m, dinv, b, W):
    N, H = m.shape
    H2 = W.shape[1]

    def body(s_ref, m_ref, dinv_ref, b_ref, w_ref, out_ref):
        t = s_ref[0] + m_ref[...]
        h = jnp.tanh(dinv_ref[...] * t + b_ref[...])
        out_ref[...] = jnp.dot(dinv_ref[...] * h, w_ref[...],
                               preferred_element_type=jnp.float32)

    return pl.pallas_call(
        body,
        grid=(N // _BR,),
        in_specs=[
            _sspec(H),
            pl.BlockSpec((_BR, H), lambda i: (i, 0)),
            pl.BlockSpec((_BR, 1), lambda i: (i, 0)),
            pl.BlockSpec((1, H), lambda i: (0, 0)),
            pl.BlockSpec((H, H2), lambda i: (0, 0)),
        ],
        out_specs=pl.BlockSpec((_BR, H2), lambda i: (i, 0)),
        out_shape=jax.ShapeDtypeStruct((N, H2), jnp.float32),
    )(s, m, dinv, b, W)


def _tc3(s, m, dinv, b):
    N, H = m.shape

    def body(s_ref, m_ref, dinv_ref, b_ref, out_ref):
        t = s_ref[0] + m_ref[...]
        h = jnp.tanh(dinv_ref[...] * t + b_ref[...])
        out_ref[...] = dinv_ref[...] * h

    return pl.pallas_call(
        body,
        grid=(N // _BR,),
        in_specs=[
            _sspec(H),
            pl.BlockSpec((_BR, H), lambda i: (i, 0)),
            pl.BlockSpec((_BR, 1), lambda i: (i, 0)),
            pl.BlockSpec((1, H), lambda i: (0, 0)),
        ],
        out_specs=pl.BlockSpec((_BR, H), lambda i: (i, 0)),
        out_shape=jax.ShapeDtypeStruct((N, H), jnp.float32),
    )(s, m, dinv, b)


def _tc4(s, u3, dinv, Wm, bm, Wl, bl, noise):
    N, H = u3.shape
    LD = Wm.shape[1]

    def body(s_ref, u_ref, dinv_ref, wm_ref, bm_ref, wl_ref, bl_ref, n_ref,
             z_ref, mean_ref, lv_ref):
        g = dinv_ref[...] * (s_ref[0] + u_ref[...])
        mean = jnp.dot(g, wm_ref[...], preferred_element_type=jnp.float32)
        mean = mean + bm_ref[...]
        lv = jnp.dot(g, wl_ref[...], preferred_element_type=jnp.float32)
        lv = lv + bl_ref[...]
        z_ref[...] = n_ref[...] * jnp.exp(0.5 * lv) + mean
        mean_ref[...] = mean
        lv_ref[...] = lv

    return pl.pallas_call(
        body,
        grid=(N // _BR,),
        in_specs=[
            _sspec(H),
            pl.BlockSpec((_BR, H), lambda i: (i, 0)),
            pl.BlockSpec((_BR, 1), lambda i: (i, 0)),
            pl.BlockSpec((H, LD), lambda i: (0, 0)),
            pl.BlockSpec((1, LD), lambda i: (0, 0)),
            pl.BlockSpec((H, LD), lambda i: (0, 0)),
            pl.BlockSpec((1, LD), lambda i: (0, 0)),
            pl.BlockSpec((_BR, LD), lambda i: (i, 0)),
        ],
        out_specs=[
            pl.BlockSpec((_BR, LD), lambda i: (i, 0)),
            pl.BlockSpec((_BR, LD), lambda i: (i, 0)),
            pl.BlockSpec((_BR, LD), lambda i: (i, 0)),
        ],
        out_shape=[
            jax.ShapeDtypeStruct((N, LD), jnp.float32),
            jax.ShapeDtypeStruct((N, LD), jnp.float32),
            jax.ShapeDtypeStruct((N, LD), jnp.float32),
        ],
    )(s, u3, dinv, Wm, bm, Wl, bl, noise)


def kernel(x, edge_index, W1, b1, W2, b2, Wm, bm, Wl, bl, noise):
    N = x.shape[0]
    E = edge_index.shape[1]
    src = edge_index[0].astype(jnp.int32)
    dst = edge_index[1].astype(jnp.int32)

    chunks = -(-E // (NS * K))
    chunks = -(-chunks // 8) * 8
    epad = NS * chunks * K
    src2 = jnp.concatenate([src, jnp.zeros((epad - E,), jnp.int32)])
    dst2 = jnp.concatenate([dst, jnp.full((epad - E,), N, jnp.int32)])

    deg = _degrees(dst2, chunks).reshape(NC, ACCR, 1)
    m1, dinv = _tc1(deg, x, W1)
    s1 = _seg_scatter(m1, src2, dst2, chunks)
    m2 = _tc_mid(s1, m1, dinv, b1.reshape(1, -1), W2)
    s2 = _seg_scatter(m2, src2, dst2, chunks)
    u3 = _tc3(s2, m2, dinv, b2.reshape(1, -1))
    s3 = _seg_scatter(u3, src2, dst2, chunks)
    z, mean, logvar = _tc4(s3, u3, dinv, Wm, bm.reshape(1, -1),
                           Wl, bl.reshape(1, -1), noise)
    return (z, mean, logvar)
